# baseline (speedup 1.0000x reference)
import functools

import jax
import jax.numpy as jnp
from jax import lax
from jax.experimental import pallas as pl
from jax.experimental.pallas import tpu as pltpu
from jax.experimental.pallas import tpu_sc as plsc

DIM = 128
SAMP = 4096
SEQ = 50
SEQP = 56
NC = 2
NS = 16
NW = NC * NS
SPW = SAMP // NW
S_CH = 16
NCH = SPW // S_CH

_mesh = plsc.VectorSubcoreMesh(core_axis_name="c", subcore_axis_name="s")


@functools.partial(
    pl.kernel,
    mesh=_mesh,
    out_type=jax.ShapeDtypeStruct((SAMP, SEQ, DIM), jnp.float32),
    scratch_types=[
        pltpu.VMEM((S_CH * SEQP,), jnp.int32),
        pltpu.VMEM((S_CH, SEQP, DIM), jnp.float32),
        pltpu.SemaphoreType.DMA,
        pltpu.SemaphoreType.DMA,
    ],
)
def _gather_kernel(idx_hbm, table_hbm, out_hbm, idx_v, rows_v, gsem, ssem):
    wid = lax.axis_index("s") * NC + lax.axis_index("c")
    base_s = wid * SPW

    def chunk(c, carry):
        s0 = base_s + c * S_CH
        pltpu.sync_copy(idx_hbm.at[pl.ds(s0 * SEQP, S_CH * SEQP)], idx_v)
        gathers = [
            pltpu.async_copy(
                table_hbm.at[idx_v.at[pl.ds(k * SEQP, SEQP)]],
                rows_v.at[k],
                gsem,
            )
            for k in range(S_CH)
        ]
        outs = []
        for k in range(S_CH):
            gathers[k].wait()
            outs.append(
                pltpu.async_copy(
                    rows_v.at[k, pl.ds(0, SEQ)],
                    out_hbm.at[s0 + k],
                    ssem,
                )
            )
        for cp in outs:
            cp.wait()
        return carry

    lax.fori_loop(0, NCH, chunk, 0)


def kernel(input, emb_weight):
    idxp = jnp.pad(input.astype(jnp.int32), ((0, 0), (0, SEQP - SEQ)))
    out = _gather_kernel(idxp.reshape(SAMP * SEQP), emb_weight)
    return out


# speedup vs baseline: 1.0131x; 1.0131x over previous
import functools

import jax
import jax.numpy as jnp
from jax import lax
from jax.experimental import pallas as pl
from jax.experimental.pallas import tpu as pltpu
from jax.experimental.pallas import tpu_sc as plsc

DIM = 128
SAMP = 4096
SEQ = 50
SEQP = 56
NC = 2
NS = 16
NW = NC * NS
SPW = SAMP // NW
S_CH = 16
NCH = SPW // S_CH

_mesh = plsc.VectorSubcoreMesh(core_axis_name="c", subcore_axis_name="s")


@functools.partial(
    pl.kernel,
    mesh=_mesh,
    out_type=jax.ShapeDtypeStruct((SAMP, SEQ, DIM), jnp.float32),
    scratch_types=[
        pltpu.VMEM((S_CH * SEQP,), jnp.int32),
        pltpu.VMEM((S_CH, SEQP, DIM), jnp.float32),
        pltpu.SemaphoreType.DMA,
        pltpu.SemaphoreType.DMA,
    ],
)
def _gather_kernel(idx_hbm, table_hbm, out_hbm, idx_v, rows_v, gsem, ssem):
    wid = lax.axis_index("s") * NC + lax.axis_index("c")
    base_s = wid * SPW

    def chunk(c, carry):
        s0 = base_s + c * S_CH
        pltpu.sync_copy(idx_hbm.at[pl.ds(s0 * SEQP, S_CH * SEQP)], idx_v)
        gathers = [
            pltpu.async_copy(
                table_hbm.at[idx_v.at[pl.ds(k * SEQP, SEQP)]],
                rows_v.at[k],
                gsem,
            )
            for k in range(S_CH)
        ]
        for cp in gathers:
            cp.wait()
        pltpu.async_copy(
            rows_v.at[pl.ds(0, S_CH), pl.ds(0, SEQ)],
            out_hbm.at[pl.ds(s0, S_CH)],
            ssem,
        ).wait()
        return carry

    lax.fori_loop(0, NCH, chunk, 0)


def kernel(input, emb_weight):
    idxp = jnp.pad(input.astype(jnp.int32), ((0, 0), (0, SEQP - SEQ)))
    out = _gather_kernel(idxp.reshape(SAMP * SEQP), emb_weight)
    return out


# speedup vs baseline: 1.0142x; 1.0010x over previous
import functools

import jax
import jax.numpy as jnp
from jax import lax
from jax.experimental import pallas as pl
from jax.experimental.pallas import tpu as pltpu
from jax.experimental.pallas import tpu_sc as plsc

DIM = 128
SAMP = 4096
SEQ = 50
SEQP = 56
NC = 2
NS = 16
NW = NC * NS
SPW = SAMP // NW
S_CH = 16
NCH = SPW // S_CH

_mesh = plsc.VectorSubcoreMesh(core_axis_name="c", subcore_axis_name="s")


@functools.partial(
    pl.kernel,
    mesh=_mesh,
    out_type=jax.ShapeDtypeStruct((SAMP, SEQ, DIM), jnp.float32),
    scratch_types=[
        pltpu.VMEM((S_CH * SEQP,), jnp.int32),
        pltpu.VMEM((S_CH * SEQP, DIM), jnp.float32),
        pltpu.SemaphoreType.DMA,
        pltpu.SemaphoreType.DMA,
    ],
)
def _gather_kernel(idx_hbm, table_hbm, out_hbm, idx_v, rows_v, gsem, ssem):
    wid = lax.axis_index("s") * NC + lax.axis_index("c")
    base_s = wid * SPW

    def chunk(c, carry):
        s0 = base_s + c * S_CH
        pltpu.sync_copy(idx_hbm.at[pl.ds(s0 * SEQP, S_CH * SEQP)], idx_v)
        gathers = [
            pltpu.async_copy(
                table_hbm.at[idx_v.at[pl.ds(j * 2 * SEQP, 2 * SEQP)]],
                rows_v.at[pl.ds(j * 2 * SEQP, 2 * SEQP)],
                gsem,
            )
            for j in range(S_CH // 2)
        ]
        for cp in gathers:
            cp.wait()
        outs = [
            pltpu.async_copy(
                rows_v.at[pl.ds(k * SEQP, SEQ)],
                out_hbm.at[s0 + k],
                ssem,
            )
            for k in range(S_CH)
        ]
        for cp in outs:
            cp.wait()
        return carry

    lax.fori_loop(0, NCH, chunk, 0)


def kernel(input, emb_weight):
    idxp = jnp.pad(input.astype(jnp.int32), ((0, 0), (0, SEQP - SEQ)))
    out = _gather_kernel(idxp.reshape(SAMP * SEQP), emb_weight)
    return out


# speedup vs baseline: 7.2884x; 7.1867x over previous
import functools

import jax
import jax.numpy as jnp
from jax import lax
from jax.experimental import pallas as pl
from jax.experimental.pallas import tpu as pltpu
from jax.experimental.pallas import tpu_sc as plsc

DIM = 128
SAMP = 4096
SEQ = 50
SEQP = 56
NPAD = SEQP - SEQ
NC = 2
NS = 16
NW = NC * NS
SPW = SAMP // NW
S_CH = 16
NCH = SPW // S_CH

_mesh = plsc.VectorSubcoreMesh(core_axis_name="c", subcore_axis_name="s")


@functools.partial(
    pl.kernel,
    mesh=_mesh,
    out_type=jax.ShapeDtypeStruct((SAMP, SEQ, DIM), jnp.float32),
    scratch_types=[
        pltpu.VMEM((S_CH * SEQP,), jnp.int32),
        pltpu.VMEM((S_CH, SEQP, DIM), jnp.float32),
        pltpu.SemaphoreType.DMA,
        pltpu.SemaphoreType.DMA,
    ],
)
def _gather_kernel(idx_hbm, table_hbm, out_hbm, idx_v, rows_v, gsem, ssem):
    wid = lax.axis_index("s") * NC + lax.axis_index("c")
    base_s = wid * SPW

    def chunk(c, carry):
        s0 = base_s + c * S_CH
        pltpu.sync_copy(idx_hbm.at[pl.ds(s0 * SEQP, S_CH * SEQP)], idx_v)
        gathers = [
            pltpu.async_copy(
                table_hbm.at[idx_v.at[pl.ds(k * SEQP, SEQP)]],
                rows_v.at[k],
                gsem,
            )
            for k in range(S_CH)
        ]
        for cp in gathers:
            cp.wait()
        pltpu.async_copy(
            rows_v.at[pl.ds(0, S_CH), pl.ds(0, SEQ)],
            out_hbm.at[pl.ds(s0, S_CH)],
            ssem,
        ).wait()
        return carry

    lax.fori_loop(0, NCH, chunk, 0)


def kernel(input, emb_weight):
    vocab = emb_weight.shape[0]
    pad = (jnp.arange(SAMP, dtype=jnp.int32)[:, None] * 8
           + jnp.arange(NPAD, dtype=jnp.int32)[None, :]) % vocab
    idxp = jnp.concatenate([input.astype(jnp.int32), pad], axis=1)
    return _gather_kernel(idxp.reshape(SAMP * SEQP), emb_weight)


# speedup vs baseline: 7.4615x; 1.0237x over previous
import functools

import jax
import jax.numpy as jnp
from jax import lax
from jax.experimental import pallas as pl
from jax.experimental.pallas import tpu as pltpu
from jax.experimental.pallas import tpu_sc as plsc

DIM = 128
SAMP = 4096
SEQ = 50
SEQP = 56
NC = 2
NS = 16
NW = NC * NS
SPW = SAMP // NW
S_CH = 8
NPAIR = SPW // (2 * S_CH)

_mesh = plsc.VectorSubcoreMesh(core_axis_name="c", subcore_axis_name="s")


@functools.partial(
    pl.kernel,
    mesh=_mesh,
    out_type=jax.ShapeDtypeStruct((SAMP, SEQ, DIM), jnp.float32),
    scratch_types=[
        pltpu.VMEM((S_CH * SEQP,), jnp.int32),
        pltpu.VMEM((S_CH * SEQP,), jnp.int32),
        pltpu.VMEM((S_CH, SEQP, DIM), jnp.float32),
        pltpu.VMEM((S_CH, SEQP, DIM), jnp.float32),
        pltpu.SemaphoreType.DMA,
        pltpu.SemaphoreType.DMA,
        pltpu.SemaphoreType.DMA,
        pltpu.SemaphoreType.DMA,
    ],
)
def _gather_kernel(idx_hbm, table_hbm, out_hbm, idx_a, idx_b, rows_a,
                   rows_b, gsem_a, gsem_b, ssem_a, ssem_b):
    wid = lax.axis_index("s") * NC + lax.axis_index("c")
    base_s = wid * SPW

    def stage(s0, idx_v, rows_v, gsem):
        pltpu.sync_copy(idx_hbm.at[pl.ds(s0 * SEQP, S_CH * SEQP)], idx_v)
        return [
            pltpu.async_copy(
                table_hbm.at[idx_v.at[pl.ds(k * SEQP, SEQ)]],
                rows_v.at[k, pl.ds(0, SEQ)],
                gsem,
            )
            for k in range(S_CH)
        ]

    def writeback(s0, rows_v, ssem):
        return pltpu.async_copy(
            rows_v.at[pl.ds(0, S_CH), pl.ds(0, SEQ)],
            out_hbm.at[pl.ds(s0, S_CH)],
            ssem,
        )

    def pair(g, carry):
        sa = base_s + g * (2 * S_CH)
        sb = sa + S_CH
        ga = stage(sa, idx_a, rows_a, gsem_a)
        gb = stage(sb, idx_b, rows_b, gsem_b)
        for cp in ga:
            cp.wait()
        wa = writeback(sa, rows_a, ssem_a)
        for cp in gb:
            cp.wait()
        wb = writeback(sb, rows_b, ssem_b)
        wa.wait()
        wb.wait()
        return carry

    lax.fori_loop(0, NPAIR, pair, 0)


def kernel(input, emb_weight):
    idxp = jnp.pad(input.astype(jnp.int32), ((0, 0), (0, SEQP - SEQ)))
    return _gather_kernel(idxp.reshape(SAMP * SEQP), emb_weight)


# speedup vs baseline: 7.4815x; 1.0027x over previous
import functools

import jax
import jax.numpy as jnp
from jax import lax
from jax.experimental import pallas as pl
from jax.experimental.pallas import tpu as pltpu
from jax.experimental.pallas import tpu_sc as plsc

DIM = 128
SAMP = 4096
SEQ = 50
SEQP = 56
NC = 2
NS = 16
NW = NC * NS
SPW = SAMP // NW
S_CH = 8
NPAIR = SPW // (2 * S_CH)

_mesh = plsc.VectorSubcoreMesh(core_axis_name="c", subcore_axis_name="s")


@functools.partial(
    pl.kernel,
    mesh=_mesh,
    out_type=jax.ShapeDtypeStruct((SAMP, SEQ, DIM), jnp.float32),
    scratch_types=[
        pltpu.VMEM((S_CH * SEQP,), jnp.int32),
        pltpu.VMEM((S_CH * SEQP,), jnp.int32),
        pltpu.VMEM((S_CH, SEQP, DIM), jnp.float32),
        pltpu.VMEM((S_CH, SEQP, DIM), jnp.float32),
        pltpu.SemaphoreType.DMA,
        pltpu.SemaphoreType.DMA,
        pltpu.SemaphoreType.DMA,
        pltpu.SemaphoreType.DMA,
    ],
)
def _gather_kernel(idx_hbm, table_hbm, out_hbm, idx_a, idx_b, rows_a,
                   rows_b, gsem_a, gsem_b, ssem_a, ssem_b):
    wid = lax.axis_index("c") * NS + lax.axis_index("s")
    base_s = wid * SPW

    def stage(s0, idx_v, rows_v, gsem):
        pltpu.sync_copy(idx_hbm.at[pl.ds(s0 * SEQP, S_CH * SEQP)], idx_v)
        return [
            pltpu.async_copy(
                table_hbm.at[idx_v.at[pl.ds(k * SEQP, SEQ)]],
                rows_v.at[k, pl.ds(0, SEQ)],
                gsem,
            )
            for k in range(S_CH)
        ]

    def writeback(s0, rows_v, ssem):
        return pltpu.async_copy(
            rows_v.at[pl.ds(0, S_CH), pl.ds(0, SEQ)],
            out_hbm.at[pl.ds(s0, S_CH)],
            ssem,
        )

    def pair(g, carry):
        sa = base_s + g * (2 * S_CH)
        sb = sa + S_CH
        ga = stage(sa, idx_a, rows_a, gsem_a)
        gb = stage(sb, idx_b, rows_b, gsem_b)
        for cp in ga:
            cp.wait()
        wa = writeback(sa, rows_a, ssem_a)
        for cp in gb:
            cp.wait()
        wb = writeback(sb, rows_b, ssem_b)
        wa.wait()
        wb.wait()
        return carry

    lax.fori_loop(0, NPAIR, pair, 0)


def kernel(input, emb_weight):
    idxp = jnp.pad(input.astype(jnp.int32), ((0, 0), (0, SEQP - SEQ)))
    return _gather_kernel(idxp.reshape(SAMP * SEQP), emb_weight)


# speedup vs baseline: 7.5253x; 1.0059x over previous
import functools

import jax
import jax.numpy as jnp
from jax import lax
from jax.experimental import pallas as pl
from jax.experimental.pallas import tpu as pltpu
from jax.experimental.pallas import tpu_sc as plsc

DIM = 128
SAMP = 4096
SEQ = 50
SEQP = 56
NC = 2
NS = 16
NW = NC * NS
SPW = SAMP // NW
S_CH = 8
NPAIR = SPW // (2 * S_CH)

_mesh = plsc.VectorSubcoreMesh(core_axis_name="c", subcore_axis_name="s")


@functools.partial(
    pl.kernel,
    mesh=_mesh,
    out_type=jax.ShapeDtypeStruct((SAMP, SEQ, DIM), jnp.float32),
    scratch_types=[
        pltpu.VMEM((S_CH, SEQ), jnp.int32),
        pltpu.VMEM((S_CH, SEQ), jnp.int32),
        pltpu.VMEM((S_CH, SEQP, DIM), jnp.float32),
        pltpu.VMEM((S_CH, SEQP, DIM), jnp.float32),
        pltpu.SemaphoreType.DMA,
        pltpu.SemaphoreType.DMA,
        pltpu.SemaphoreType.DMA,
        pltpu.SemaphoreType.DMA,
    ],
)
def _gather_kernel(idx_hbm, table_hbm, out_hbm, idx_a, idx_b, rows_a,
                   rows_b, gsem_a, gsem_b, ssem_a, ssem_b):
    wid = lax.axis_index("s") * NC + lax.axis_index("c")
    base_s = wid * SPW

    def stage(s0, idx_v, rows_v, gsem):
        pltpu.sync_copy(idx_hbm.at[pl.ds(s0, S_CH)], idx_v)
        return [
            pltpu.async_copy(
                table_hbm.at[idx_v.at[k]],
                rows_v.at[k, pl.ds(0, SEQ)],
                gsem,
            )
            for k in range(S_CH)
        ]

    def writeback(s0, rows_v, ssem):
        return pltpu.async_copy(
            rows_v.at[pl.ds(0, S_CH), pl.ds(0, SEQ)],
            out_hbm.at[pl.ds(s0, S_CH)],
            ssem,
        )

    def pair(g, carry):
        sa = base_s + g * (2 * S_CH)
        sb = sa + S_CH
        ga = stage(sa, idx_a, rows_a, gsem_a)
        gb = stage(sb, idx_b, rows_b, gsem_b)
        for cp in ga:
            cp.wait()
        wa = writeback(sa, rows_a, ssem_a)
        for cp in gb:
            cp.wait()
        wb = writeback(sb, rows_b, ssem_b)
        wa.wait()
        wb.wait()
        return carry

    lax.fori_loop(0, NPAIR, pair, 0)


def kernel(input, emb_weight):
    return _gather_kernel(input.astype(jnp.int32), emb_weight)
